# baseline (speedup 1.0000x reference)
import functools

import jax
import jax.numpy as jnp
from jax import lax
from jax.experimental import pallas as pl
from jax.experimental.pallas import tpu as pltpu
from jax.experimental.pallas import tpu_sc as plsc

E = 320000
D = 128
VOCAB_ROWS = 50
NUM_FEAT = 3
NC = 2
NS = 16
NW = NC * NS
BPW = E // NW
C = 80
NCH = BPW // C
LANES = 16
GROUPS = C // LANES
TBL = VOCAB_ROWS * D

_mesh = plsc.VectorSubcoreMesh(core_axis_name="c", subcore_axis_name="s")


@functools.partial(
    pl.kernel,
    mesh=_mesh,
    compiler_params=pltpu.CompilerParams(needs_layout_passes=False),
    out_type=jax.ShapeDtypeStruct((E * D,), jnp.float32),
    scratch_types=[
        pltpu.VMEM((NCH, C), jnp.int32),
        pltpu.VMEM((NCH, C), jnp.int32),
        pltpu.VMEM((NCH, C), jnp.int32),
        pltpu.VMEM((TBL,), jnp.float32),
        pltpu.VMEM((TBL,), jnp.float32),
        pltpu.VMEM((TBL,), jnp.float32),
        pltpu.VMEM((C * D,), jnp.float32),
        pltpu.SemaphoreType.DMA,
    ],
)
def _bond_encode(idx_hbm, e0, e1, e2, out, idx0_v, idx1_v, idx2_v,
                 t0, t1, t2, ob, sem):
    cid = lax.axis_index("c")
    sid = lax.axis_index("s")
    wid = sid * NC + cid

    pltpu.sync_copy(e0, t0)
    pltpu.sync_copy(e1, t1)
    pltpu.sync_copy(e2, t2)
    pltpu.sync_copy(idx_hbm.at[0, wid], idx0_v)
    pltpu.sync_copy(idx_hbm.at[1, wid], idx1_v)
    pltpu.sync_copy(idx_hbm.at[2, wid], idx2_v)

    iota128 = lax.iota(jnp.int32, LANES) * D

    def chunk_body(i, carry):
        for g in range(GROUPS):
            s = pl.ds(g * LANES, LANES)
            b0 = idx0_v[i, s] * D
            b1 = idx1_v[i, s] * D
            b2 = idx2_v[i, s] * D
            ov = iota128 + (g * LANES * D)

            def col_body(c, bases):
                cb0, cb1, cb2, cov = bases
                v0 = plsc.load_gather(t0, [cb0])
                v1 = plsc.load_gather(t1, [cb1])
                v2 = plsc.load_gather(t2, [cb2])
                plsc.store_scatter(ob, [cov], v0 + v1 + v2)
                return (cb0 + 1, cb1 + 1, cb2 + 1, cov + 1)

            lax.fori_loop(0, D, col_body, (b0, b1, b2, ov), unroll=8)

        pltpu.sync_copy(ob, out.at[pl.ds((wid * BPW + i * C) * D, C * D)])
        return carry

    lax.fori_loop(0, NCH, chunk_body, 0)


def kernel(edge_attr, emb0, emb1, emb2):
    idx = edge_attr.astype(jnp.int32).T.reshape(NUM_FEAT, NW, NCH, C)
    flat = _bond_encode(idx, emb0.reshape(-1), emb1.reshape(-1),
                        emb2.reshape(-1))
    return flat.reshape(E, D)


# speedup vs baseline: 1.8795x; 1.8795x over previous
import functools

import jax
import jax.numpy as jnp
from jax import lax
from jax.experimental import pallas as pl
from jax.experimental.pallas import tpu as pltpu
from jax.experimental.pallas import tpu_sc as plsc

E = 320000
D = 128
VOCAB_ROWS = 50
NUM_FEAT = 3
NC = 2
NS = 16
NW = NC * NS
BPW = E // NW
C = 80
NCH = BPW // C
LANES = 16
GROUPS = C // LANES
TBL = VOCAB_ROWS * D

_mesh = plsc.VectorSubcoreMesh(core_axis_name="c", subcore_axis_name="s")


@functools.partial(
    pl.kernel,
    mesh=_mesh,
    compiler_params=pltpu.CompilerParams(needs_layout_passes=False),
    out_type=jax.ShapeDtypeStruct((E * D,), jnp.float32),
    scratch_types=[
        pltpu.VMEM((NCH, C), jnp.int32),
        pltpu.VMEM((NCH, C), jnp.int32),
        pltpu.VMEM((NCH, C), jnp.int32),
        pltpu.VMEM((TBL,), jnp.float32),
        pltpu.VMEM((TBL,), jnp.float32),
        pltpu.VMEM((TBL,), jnp.float32),
        pltpu.VMEM((C * D,), jnp.float32),
        pltpu.VMEM((C * D,), jnp.float32),
        pltpu.SemaphoreType.DMA,
        pltpu.SemaphoreType.DMA,
    ],
)
def _bond_encode(idx_hbm, e0, e1, e2, out, idx0_v, idx1_v, idx2_v,
                 t0, t1, t2, ob_a, ob_b, sem_a, sem_b):
    cid = lax.axis_index("c")
    sid = lax.axis_index("s")
    wid = sid * NC + cid

    pltpu.sync_copy(e0, t0)
    pltpu.sync_copy(e1, t1)
    pltpu.sync_copy(e2, t2)
    pltpu.sync_copy(idx_hbm.at[0, wid], idx0_v)
    pltpu.sync_copy(idx_hbm.at[1, wid], idx1_v)
    pltpu.sync_copy(idx_hbm.at[2, wid], idx2_v)

    iota128 = lax.iota(jnp.int32, LANES) * D

    def fill(i, ob):
        for g in range(GROUPS):
            s = pl.ds(g * LANES, LANES)
            b0 = idx0_v[i, s] * D
            b1 = idx1_v[i, s] * D
            b2 = idx2_v[i, s] * D
            ov = iota128 + (g * LANES * D)

            @plsc.parallel_loop(0, D, unroll=8)
            def _cols(c):
                v0 = plsc.load_gather(t0, [b0 + c])
                v1 = plsc.load_gather(t1, [b1 + c])
                v2 = plsc.load_gather(t2, [b2 + c])
                plsc.store_scatter(ob, [ov + c], v0 + v1 + v2)

    def start_wb(i, ob, sem):
        pltpu.async_copy(ob, out.at[pl.ds((wid * BPW + i * C) * D, C * D)],
                         sem)

    def drain_wb(ob, sem):
        pltpu.make_async_copy(ob, out.at[pl.ds(wid * BPW * D, C * D)],
                              sem).wait()

    def pair_body(j, carry):
        a = 2 * j

        @pl.when(j > 0)
        def _():
            drain_wb(ob_a, sem_a)

        fill(a, ob_a)
        start_wb(a, ob_a, sem_a)

        @pl.when(j > 0)
        def _():
            drain_wb(ob_b, sem_b)

        fill(a + 1, ob_b)
        start_wb(a + 1, ob_b, sem_b)
        return carry

    lax.fori_loop(0, NCH // 2, pair_body, 0)

    drain_wb(ob_a, sem_a)
    fill(NCH - 1, ob_a)
    start_wb(NCH - 1, ob_a, sem_a)
    drain_wb(ob_a, sem_a)
    drain_wb(ob_b, sem_b)


def kernel(edge_attr, emb0, emb1, emb2):
    idx = edge_attr.astype(jnp.int32).T.reshape(NUM_FEAT, NW, NCH, C)
    flat = _bond_encode(idx, emb0.reshape(-1), emb1.reshape(-1),
                        emb2.reshape(-1))
    return flat.reshape(E, D)


# speedup vs baseline: 10.6071x; 5.6435x over previous
import functools

import jax
import jax.numpy as jnp
from jax import lax
from jax.experimental import pallas as pl
from jax.experimental.pallas import tpu as pltpu
from jax.experimental.pallas import tpu_sc as plsc

E = 320000
D = 128
VOCAB_ROWS = 50
NUM_FEAT = 3
NC = 2
NS = 16
NW = NC * NS
BPW = E // NW
C = 80
NCH = BPW // C
LANES = 16
GROUPS = C // LANES
TBL = VOCAB_ROWS * D
ROW_SLICES = D // LANES

_mesh = plsc.VectorSubcoreMesh(core_axis_name="c", subcore_axis_name="s")


@functools.partial(
    pl.kernel,
    mesh=_mesh,
    compiler_params=pltpu.CompilerParams(needs_layout_passes=False),
    out_type=jax.ShapeDtypeStruct((E, D), jnp.float32),
    scratch_types=[
        pltpu.VMEM((NCH, C), jnp.int32),
        pltpu.VMEM((NCH, C), jnp.int32),
        pltpu.VMEM((NCH, C), jnp.int32),
        pltpu.VMEM((TBL,), jnp.float32),
        pltpu.VMEM((TBL,), jnp.float32),
        pltpu.VMEM((TBL,), jnp.float32),
        pltpu.VMEM((C, D), jnp.float32),
        pltpu.VMEM((C, D), jnp.float32),
        pltpu.SemaphoreType.DMA,
        pltpu.SemaphoreType.DMA,
    ],
)
def _bond_encode(idx_hbm, e0, e1, e2, out, idx0_v, idx1_v, idx2_v,
                 t0, t1, t2, ob_a, ob_b, sem_a, sem_b):
    cid = lax.axis_index("c")
    sid = lax.axis_index("s")
    wid = sid * NC + cid

    pltpu.sync_copy(e0, t0)
    pltpu.sync_copy(e1, t1)
    pltpu.sync_copy(e2, t2)
    pltpu.sync_copy(idx_hbm.at[0, wid], idx0_v)
    pltpu.sync_copy(idx_hbm.at[1, wid], idx1_v)
    pltpu.sync_copy(idx_hbm.at[2, wid], idx2_v)

    iotas = [lax.iota(jnp.int32, LANES) + k * LANES for k in range(ROW_SLICES)]

    def fill(i, ob):
        for g in range(GROUPS):
            s = pl.ds(g * LANES, LANES)
            b0 = idx0_v[i, s] * D
            b1 = idx1_v[i, s] * D
            b2 = idx2_v[i, s] * D

            @plsc.parallel_loop(0, LANES, unroll=2)
            def _edges(l):
                lane = jnp.full((LANES,), l, jnp.int32)
                e0b = b0.at[lane].get(mode="promise_in_bounds")
                e1b = b1.at[lane].get(mode="promise_in_bounds")
                e2b = b2.at[lane].get(mode="promise_in_bounds")
                row = g * LANES + l
                for k in range(ROW_SLICES):
                    v0 = plsc.load_gather(t0, [e0b + iotas[k]])
                    v1 = plsc.load_gather(t1, [e1b + iotas[k]])
                    v2 = plsc.load_gather(t2, [e2b + iotas[k]])
                    ob[row, pl.ds(k * LANES, LANES)] = v0 + v1 + v2

    def start_wb(i, ob, sem):
        pltpu.async_copy(ob, out.at[pl.ds(wid * BPW + i * C, C)], sem)

    def drain_wb(ob, sem):
        pltpu.make_async_copy(ob, out.at[pl.ds(wid * BPW, C)], sem).wait()

    def pair_body(j, carry):
        a = 2 * j

        @pl.when(j > 0)
        def _():
            drain_wb(ob_a, sem_a)

        fill(a, ob_a)
        start_wb(a, ob_a, sem_a)

        @pl.when(j > 0)
        def _():
            drain_wb(ob_b, sem_b)

        fill(a + 1, ob_b)
        start_wb(a + 1, ob_b, sem_b)
        return carry

    lax.fori_loop(0, NCH // 2, pair_body, 0)

    drain_wb(ob_a, sem_a)
    fill(NCH - 1, ob_a)
    start_wb(NCH - 1, ob_a, sem_a)
    drain_wb(ob_a, sem_a)
    drain_wb(ob_b, sem_b)


def kernel(edge_attr, emb0, emb1, emb2):
    idx = edge_attr.astype(jnp.int32).T.reshape(NUM_FEAT, NW, NCH, C)
    return _bond_encode(idx, emb0.reshape(-1), emb1.reshape(-1),
                        emb2.reshape(-1))
